# baseline (speedup 1.0000x reference)
import functools

import jax
import jax.numpy as jnp
from jax import lax
from jax.experimental import pallas as pl
from jax.experimental.pallas import tpu as pltpu
from jax.experimental.pallas import tpu_sc as plsc

N, K, C = 10000, 27, 256
KP = 28
RB = 4
NC, NS = 2, 16
NW = NC * NS
NCHUNK = (N + RB - 1) // RB
CPW = -(-NCHUNK // NW)
CPW += CPW % 2
NPAD = NW * CPW * RB
IDX_W = CPW * RB * KP
ZR = N
LANES = 16


def _dwconv_body(data_hbm, nflat_hbm, w_hbm, out_hbm,
                 idx_v, g0, g1, w_v, out_v, sem0, sem1):
    wid = lax.axis_index("c") * NS + lax.axis_index("s")
    base0 = wid * CPW
    nvalid = jnp.minimum(CPW, jnp.maximum(NCHUNK - base0, 0))

    pltpu.sync_copy(w_hbm, w_v)
    pltpu.sync_copy(nflat_hbm.at[pl.ds(wid * IDX_W, IDX_W)], idx_v)

    def _clean(i, carry):
        v = idx_v[pl.ds(i * LANES, LANES)]
        idx_v[pl.ds(i * LANES, LANES)] = jnp.where(v < 0, ZR, v)
        return carry
    lax.fori_loop(0, IDX_W // LANES, _clean, 0)

    def _gather(j, gbuf, sem):
        return pltpu.make_async_copy(
            data_hbm.at[idx_v.at[pl.ds(j * (RB * KP), RB * KP)]], gbuf, sem)

    def _compute(gbuf):
        def c_body(ci, carry):
            cs = ci * LANES
            wr = [w_v[k, pl.ds(cs, LANES)] for k in range(K)]
            for r in range(RB):
                acc = gbuf[r * KP, pl.ds(cs, LANES)] * wr[0]
                for k in range(1, K):
                    acc = acc + gbuf[r * KP + k, pl.ds(cs, LANES)] * wr[k]
                out_v[r, pl.ds(cs, LANES)] = acc
            return carry
        lax.fori_loop(0, C // LANES, c_body, 0)

    @pl.when(nvalid > 0)
    def _prime0():
        _gather(0, g0, sem0).start()

    @pl.when(nvalid > 1)
    def _prime1():
        _gather(1, g1, sem1).start()

    def _pair(j2, carry):
        for b, (gbuf, sem) in enumerate(((g0, sem0), (g1, sem1))):
            j = j2 * 2 + b

            @pl.when(j < nvalid)
            def _do():
                _gather(j, gbuf, sem).wait()
                _compute(gbuf)
                pltpu.sync_copy(out_v, out_hbm.at[pl.ds((base0 + j) * RB, RB)])

                @pl.when(j + 2 < nvalid)
                def _next():
                    _gather(j + 2, gbuf, sem).start()
        return carry
    lax.fori_loop(0, CPW // 2, _pair, 0)


@functools.cache
def _dwconv():
    return functools.partial(
        pl.kernel,
        out_type=jax.ShapeDtypeStruct((N, C), jnp.float32),
        mesh=plsc.VectorSubcoreMesh(core_axis_name="c", subcore_axis_name="s",
                                    num_cores=NC, num_subcores=NS),
        scratch_types=[
            pltpu.VMEM((IDX_W,), jnp.int32),
            pltpu.VMEM((RB * KP, C), jnp.float32),
            pltpu.VMEM((RB * KP, C), jnp.float32),
            pltpu.VMEM((K, C), jnp.float32),
            pltpu.VMEM((RB, C), jnp.float32),
            pltpu.SemaphoreType.DMA,
            pltpu.SemaphoreType.DMA,
        ],
    )(_dwconv_body)


def kernel(data, neigh, weights):
    data_p = jnp.concatenate([data, jnp.zeros((8, C), jnp.float32)], axis=0)
    n28 = jnp.concatenate(
        [neigh, jnp.full((N, KP - K), -1, jnp.int32)], axis=1)
    nflat = jnp.concatenate(
        [n28, jnp.full((NPAD - N, KP), -1, jnp.int32)], axis=0).reshape(-1)
    return _dwconv()(data_p, nflat, weights.reshape(K, C))


# speedup vs baseline: 1.0010x; 1.0010x over previous
import functools

import jax
import jax.numpy as jnp
from jax import lax
from jax.experimental import pallas as pl
from jax.experimental.pallas import tpu as pltpu
from jax.experimental.pallas import tpu_sc as plsc

N, K, C = 10000, 27, 256
KP = 28
RB = 4
NC, NS = 2, 16
NW = NC * NS
NCHUNK = (N + RB - 1) // RB
CPW = -(-NCHUNK // NW)
CPW += CPW % 2
NPAD = NW * CPW * RB
CW = RB * KP
ZR = N
LANES = 16


def _dwconv_body(data_hbm, nflat_hbm, w_hbm, out_hbm,
                 idx_v, g0, g1, w_v, out_v, sem0, sem1):
    wid = lax.axis_index("c") * NS + lax.axis_index("s")
    base0 = wid * CPW
    nvalid = jnp.minimum(CPW, jnp.maximum(NCHUNK - base0, 0))

    pltpu.sync_copy(w_hbm, w_v)
    pltpu.sync_copy(nflat_hbm.at[pl.ds(wid * CPW, CPW)], idx_v)

    def _clean(i, carry):
        for u in range(CW // LANES):
            v = idx_v[i, pl.ds(u * LANES, LANES)]
            idx_v[i, pl.ds(u * LANES, LANES)] = jnp.where(v < 0, ZR, v)
        return carry
    lax.fori_loop(0, CPW, _clean, 0)

    def _gather(j, gbuf, sem):
        return pltpu.make_async_copy(data_hbm.at[idx_v.at[j]], gbuf, sem)

    def _compute(gbuf):
        def c_body(ci, carry):
            cs = ci * LANES
            wr = [w_v[k, pl.ds(cs, LANES)] for k in range(K)]
            for r in range(RB):
                acc = gbuf[r * KP, pl.ds(cs, LANES)] * wr[0]
                for k in range(1, K):
                    acc = acc + gbuf[r * KP + k, pl.ds(cs, LANES)] * wr[k]
                out_v[r, pl.ds(cs, LANES)] = acc
            return carry
        lax.fori_loop(0, C // LANES, c_body, 0)

    @pl.when(nvalid > 0)
    def _prime0():
        _gather(0, g0, sem0).start()

    @pl.when(nvalid > 1)
    def _prime1():
        _gather(1, g1, sem1).start()

    def _pair(j2, carry):
        for b, (gbuf, sem) in enumerate(((g0, sem0), (g1, sem1))):
            j = j2 * 2 + b

            @pl.when(j < nvalid)
            def _do():
                _gather(j, gbuf, sem).wait()
                _compute(gbuf)
                pltpu.sync_copy(out_v, out_hbm.at[pl.ds((base0 + j) * RB, RB)])

                @pl.when(j + 2 < nvalid)
                def _next():
                    _gather(j + 2, gbuf, sem).start()
        return carry
    lax.fori_loop(0, CPW // 2, _pair, 0)


@functools.cache
def _dwconv():
    return functools.partial(
        pl.kernel,
        out_type=jax.ShapeDtypeStruct((N, C), jnp.float32),
        mesh=plsc.VectorSubcoreMesh(core_axis_name="c", subcore_axis_name="s",
                                    num_cores=NC, num_subcores=NS),
        compiler_params=pltpu.CompilerParams(use_tc_tiling_on_sc=False),
        scratch_types=[
            pltpu.VMEM((CPW, CW), jnp.int32),
            pltpu.VMEM((RB * KP, C), jnp.float32),
            pltpu.VMEM((RB * KP, C), jnp.float32),
            pltpu.VMEM((K, C), jnp.float32),
            pltpu.VMEM((RB, C), jnp.float32),
            pltpu.SemaphoreType.DMA,
            pltpu.SemaphoreType.DMA,
        ],
    )(_dwconv_body)


def kernel(data, neigh, weights):
    data_p = jnp.concatenate([data, jnp.zeros((8, C), jnp.float32)], axis=0)
    n28 = jnp.concatenate(
        [neigh, jnp.full((N, KP - K), -1, jnp.int32)], axis=1)
    nflat = jnp.concatenate(
        [n28, jnp.full((NPAD - N, KP), -1, jnp.int32)], axis=0).reshape(
            NW * CPW, CW)
    return _dwconv()(data_p, nflat, weights.reshape(K, C))
